# baseline (speedup 1.0000x reference)
import functools

import jax
import jax.numpy as jnp
from jax import lax
from jax.experimental import pallas as pl
from jax.experimental.pallas import tpu as pltpu
from jax.experimental.pallas import tpu_sc as plsc

B = 4
T = 2048
D = 64
NC = 2
NS = 16
NW = NC * NS
N = B * T
CHUNK = N // NW
HALF = CHUNK // 2
LANES = 16


def _emb_body(x_hbm, wte_hbm, wtp_hbm, out_hbm, idx_v, rows_v, pos_v, sem):
    wid = lax.axis_index("s") * NC + lax.axis_index("c")
    base = wid * CHUNK
    pos_off = lax.rem(base, T)

    pltpu.sync_copy(x_hbm.at[pl.ds(base, CHUNK)], idx_v)

    cp0 = pltpu.async_copy(
        wte_hbm.at[idx_v.at[pl.ds(0, HALF)]],
        rows_v.at[pl.ds(0, HALF)], sem)
    cp1 = pltpu.async_copy(
        wte_hbm.at[idx_v.at[pl.ds(HALF, HALF)]],
        rows_v.at[pl.ds(HALF, HALF)], sem)

    pltpu.sync_copy(wtp_hbm.at[pl.ds(pos_off, CHUNK)], pos_v)
    cp0.wait()
    cp1.wait()

    def add_row(r, carry):
        for c in range(0, D, LANES):
            rows_v[r, pl.ds(c, LANES)] = (
                rows_v[r, pl.ds(c, LANES)] + pos_v[r, pl.ds(c, LANES)])
        return carry

    lax.fori_loop(0, CHUNK, add_row, 0)

    pltpu.sync_copy(rows_v, out_hbm.at[pl.ds(base, CHUNK)])


@jax.jit
def _emb_lookup(x_flat, wte, wtp):
    mesh = plsc.VectorSubcoreMesh(core_axis_name="c", subcore_axis_name="s")
    return pl.kernel(
        _emb_body,
        out_type=jax.ShapeDtypeStruct((N, D), jnp.float32),
        mesh=mesh,
        scratch_types=[
            pltpu.VMEM((CHUNK,), jnp.int32),
            pltpu.VMEM((CHUNK, D), jnp.float32),
            pltpu.VMEM((CHUNK, D), jnp.float32),
            pltpu.SemaphoreType.DMA,
        ],
        compiler_params=pltpu.CompilerParams(use_tc_tiling_on_sc=False),
    )(x_flat, wte, wtp)


def kernel(x, wte, wtp):
    out = _emb_lookup(x.reshape(-1), wte, wtp)
    return out.reshape(B, T, D)
